# Initial kernel scaffold; baseline (speedup 1.0000x reference)
import jax
import jax.numpy as jnp
from jax.experimental import pallas as pl


def kernel(node_representations, branches, branch_weights, m_g1, m_b1, m_mu1, m_v1, m_W1, m_c1, m_g2, m_b2, m_mu2, m_v2, m_W2, m_c2, e_g1, e_b1, e_mu1, e_v1, e_W1, e_c1, e_g2, e_b2, e_mu2, e_v2, e_W2, e_c2):
    raise NotImplementedError("write your pallas kernel here")


# speedup vs baseline: 4.1494x; 4.1494x over previous
import functools

import jax
import jax.numpy as jnp
from jax import lax
from jax.experimental import pallas as pl
from jax.experimental.pallas import tpu as pltpu
from jax.experimental.pallas import tpu_sc as plsc

N = 10000
E = 320000
D = 128
H = 128

NC = 2
NS = 16
L = 16

HH = H // 2
C = 80
NCHUNK = 256
EW = NCHUNK * C
EP = NS * EW
OB = 200
NBLK = N // OB


def _gelu(x):
    return x * (0.5 * (1.0 + lax.erf(x * (2.0 ** -0.5))))


def _node_ffn(x, W1, c1, W2, c2):
    blk = 1000

    def body(x_ref, w1_ref, c1_ref, w2_ref, c2_ref, o_ref):
        h = jnp.dot(x_ref[...], w1_ref[...], preferred_element_type=jnp.float32)
        h = _gelu(h + c1_ref[...])
        h = jnp.dot(h, w2_ref[...], preferred_element_type=jnp.float32)
        o_ref[...] = _gelu(h + c2_ref[...])

    return pl.pallas_call(
        body,
        grid=(N // blk,),
        in_specs=[
            pl.BlockSpec((blk, D), lambda i: (i, 0)),
            pl.BlockSpec((D, H), lambda i: (0, 0)),
            pl.BlockSpec((1, H), lambda i: (0, 0)),
            pl.BlockSpec((H, H), lambda i: (0, 0)),
            pl.BlockSpec((1, H), lambda i: (0, 0)),
        ],
        out_specs=pl.BlockSpec((blk, H), lambda i: (i, 0)),
        out_shape=jax.ShapeDtypeStruct((N, H), jnp.float32),
    )(x, W1, c1, W2, c2)


def _bcast_lane(v16, lane):
    idx = jnp.full((L, 1), lane, dtype=jnp.int32)
    return lax.gather(
        v16, idx,
        lax.GatherDimensionNumbers(
            offset_dims=(), collapsed_slice_dims=(0,), start_index_map=(0,)),
        slice_sizes=(1,),
        mode=lax.GatherScatterMode.PROMISE_IN_BOUNDS)


def _sc_segment_sum(F2, nbr2, dst2d, w):
    mesh = plsc.VectorSubcoreMesh(core_axis_name="c", subcore_axis_name="s")

    @functools.partial(
        pl.kernel,
        out_type=jax.ShapeDtypeStruct((NC, N, HH), jnp.float32),
        mesh=mesh,
        scratch_types=[
            pltpu.VMEM((NCHUNK, C), jnp.int32),
            pltpu.VMEM((NCHUNK, C), jnp.int32),
            pltpu.VMEM((EW,), jnp.float32),
            pltpu.VMEM((C, HH), jnp.float32),
            pltpu.VMEM((OB, HH), jnp.float32),
            pltpu.VMEM_SHARED((N, HH), jnp.float32),
            pltpu.SemaphoreType.DMA,
        ],
        compiler_params=pltpu.CompilerParams(use_tc_tiling_on_sc=False),
    )
    def k(f_hbm, nbr_hbm, dst_hbm, w_hbm, out_hbm,
          idx_v, dst_v, w_v, rows_v, stage_v, agg_sh, sem):
        c = lax.axis_index("c")
        s = lax.axis_index("s")

        def zero_row(r, carry):
            for q in range(HH // L):
                stage_v[r, pl.ds(q * L, L)] = jnp.zeros((L,), jnp.float32)
            return carry
        lax.fori_loop(0, OB, zero_row, 0)
        for i in range(pl.cdiv(NBLK, NS)):
            b = s + NS * i

            @pl.when(b < NBLK)
            def _():
                r0 = pl.multiple_of(b * OB, 8)
                pltpu.sync_copy(stage_v, agg_sh.at[pl.ds(r0, OB)])

        row0 = s * NCHUNK
        pltpu.sync_copy(nbr_hbm.at[c, pl.ds(row0, NCHUNK)], idx_v)
        pltpu.sync_copy(dst_hbm.at[pl.ds(row0, NCHUNK)], dst_v)
        pltpu.sync_copy(w_hbm.at[pl.ds(s * EW, EW)], w_v)

        plsc.subcore_barrier()

        def chunk(j, carry):
            pltpu.async_copy(f_hbm.at[idx_v.at[j]], rows_v, sem).wait()
            for g in range(C // L):
                w16 = w_v[pl.ds(j * C + g * L, L)]
                for lane in range(L):
                    e = g * L + lane
                    ws = _bcast_lane(w16, lane)
                    for q in range(HH // L):
                        sl = pl.ds(q * L, L)
                        rows_v[e, sl] = rows_v[e, sl] * ws
            pltpu.sync_copy(rows_v, agg_sh.at[dst_v.at[j]], add=True)
            return carry
        lax.fori_loop(0, NCHUNK, chunk, 0)

        plsc.subcore_barrier()

        for i in range(pl.cdiv(NBLK, NS)):
            b = s + NS * i

            @pl.when(b < NBLK)
            def _():
                r0 = pl.multiple_of(b * OB, 8)
                pltpu.sync_copy(agg_sh.at[pl.ds(r0, OB)], stage_v)
                pltpu.sync_copy(stage_v, out_hbm.at[c, pl.ds(r0, OB)])

    return k(F2, nbr2, dst2d, w)


def _out_ffn(x, p, Wa, Wb, c1, W2, c2):
    blk = 1000

    def body(x_ref, p_ref, wa_ref, wb_ref, c1_ref, w2_ref, c2_ref, o_ref):
        h = jnp.dot(x_ref[...], wa_ref[...], preferred_element_type=jnp.float32)
        h = h + jnp.dot(p_ref[0], wb_ref[...][:HH], preferred_element_type=jnp.float32)
        h = h + jnp.dot(p_ref[1], wb_ref[...][HH:], preferred_element_type=jnp.float32)
        h = _gelu(h + c1_ref[...])
        h = jnp.dot(h, w2_ref[...], preferred_element_type=jnp.float32)
        h = _gelu(h + c2_ref[...])
        nrm = lax.rsqrt(jnp.maximum(jnp.sum(h * h, axis=-1, keepdims=True), 1e-12))
        o_ref[...] = h * nrm

    return pl.pallas_call(
        body,
        grid=(N // blk,),
        in_specs=[
            pl.BlockSpec((blk, D), lambda i: (i, 0)),
            pl.BlockSpec((NC, blk, HH), lambda i: (0, i, 0)),
            pl.BlockSpec((D, H), lambda i: (0, 0)),
            pl.BlockSpec((H, H), lambda i: (0, 0)),
            pl.BlockSpec((1, H), lambda i: (0, 0)),
            pl.BlockSpec((H, H), lambda i: (0, 0)),
            pl.BlockSpec((1, H), lambda i: (0, 0)),
        ],
        out_specs=pl.BlockSpec((blk, H), lambda i: (i, 0)),
        out_shape=jax.ShapeDtypeStruct((N, H), jnp.float32),
    )(x, p, Wa, Wb, c1, W2, c2)


def kernel(node_representations, branches, branch_weights,
           m_g1, m_b1, m_mu1, m_v1, m_W1, m_c1,
           m_g2, m_b2, m_mu2, m_v2, m_W2, m_c2,
           e_g1, e_b1, e_mu1, e_v1, e_W1, e_c1,
           e_g2, e_b2, e_mu2, e_v2, e_W2, e_c2):
    x = node_representations[0]
    dst = branches[0]
    nbr = branches[1]
    w = branch_weights[0, :, 0]

    s1 = m_g1 * lax.rsqrt(m_v1 + 1e-3)
    t1 = m_b1 - m_mu1 * s1
    mW1 = s1[:, None] * m_W1
    mc1 = (m_c1 + t1 @ m_W1)[None]
    s2 = m_g2 * lax.rsqrt(m_v2 + 1e-3)
    t2 = m_b2 - m_mu2 * s2
    mW2 = s2[:, None] * m_W2
    mc2 = (m_c2 + t2 @ m_W2)[None]

    F = _node_ffn(x, mW1, mc1, mW2, mc2)

    zpad_i = jnp.zeros((EP - E,), jnp.int32)
    zpad_f = jnp.zeros((EP - E,), jnp.float32)
    nbr_p = jnp.concatenate([nbr, zpad_i])
    nbr2 = jnp.stack([nbr_p, nbr_p + N]).reshape(NC, EP // C, C)
    dst_p = jnp.concatenate([dst, zpad_i]).reshape(EP // C, C)
    w_p = jnp.concatenate([w, zpad_f])
    F2 = jnp.concatenate([F[:, :HH], F[:, HH:]], axis=0)
    p = _sc_segment_sum(F2, nbr2, dst_p, w_p)

    se = e_g1 * lax.rsqrt(e_v1 + 1e-3)
    te = e_b1 - e_mu1 * se
    eW1 = se[:, None] * e_W1
    ec1 = (e_c1 + te @ e_W1)[None]
    sf = e_g2 * lax.rsqrt(e_v2 + 1e-3)
    tf = e_b2 - e_mu2 * sf
    eW2 = sf[:, None] * e_W2
    ec2 = (e_c2 + tf @ e_W2)[None]

    out = _out_ffn(x, p, eW1[:D], eW1[D:], ec1, eW2, ec2)
    return out[None]


# speedup vs baseline: 6.0752x; 1.4641x over previous
import functools

import jax
import jax.numpy as jnp
from jax import lax
from jax.experimental import pallas as pl
from jax.experimental.pallas import tpu as pltpu
from jax.experimental.pallas import tpu_sc as plsc

N = 10000
E = 320000
D = 128
H = 128

NC = 2
NS = 16
L = 16

HH = H // 2
NBUF = 4
C = 80
NCHUNK = 256
EW = NCHUNK * C
EP = NS * EW
OB = C
NBLK = N // OB


def _gelu(x):
    return x * (0.5 * (1.0 + lax.erf(x * (2.0 ** -0.5))))


def _node_ffn(x, W1, c1, W2, c2):
    blk = 1000

    def body(x_ref, w1_ref, c1_ref, w2_ref, c2_ref, o_ref):
        h = jnp.dot(x_ref[...], w1_ref[...], preferred_element_type=jnp.float32)
        h = _gelu(h + c1_ref[...])
        h = jnp.dot(h, w2_ref[...], preferred_element_type=jnp.float32)
        o_ref[...] = _gelu(h + c2_ref[...])

    return pl.pallas_call(
        body,
        grid=(N // blk,),
        in_specs=[
            pl.BlockSpec((blk, D), lambda i: (i, 0)),
            pl.BlockSpec((D, H), lambda i: (0, 0)),
            pl.BlockSpec((1, H), lambda i: (0, 0)),
            pl.BlockSpec((H, H), lambda i: (0, 0)),
            pl.BlockSpec((1, H), lambda i: (0, 0)),
        ],
        out_specs=pl.BlockSpec((blk, H), lambda i: (i, 0)),
        out_shape=jax.ShapeDtypeStruct((N, H), jnp.float32),
    )(x, W1, c1, W2, c2)


def _bcast_lane(v16, lane):
    idx = jnp.full((L, 1), lane, dtype=jnp.int32)
    return lax.gather(
        v16, idx,
        lax.GatherDimensionNumbers(
            offset_dims=(), collapsed_slice_dims=(0,), start_index_map=(0,)),
        slice_sizes=(1,),
        mode=lax.GatherScatterMode.PROMISE_IN_BOUNDS)


def _sc_segment_sum(F2, nbr2, dst2d, w):
    mesh = plsc.VectorSubcoreMesh(core_axis_name="c", subcore_axis_name="s")

    @functools.partial(
        pl.kernel,
        out_type=jax.ShapeDtypeStruct((NC, N, HH), jnp.float32),
        mesh=mesh,
        scratch_types=[
            pltpu.VMEM((NCHUNK, C), jnp.int32),
            pltpu.VMEM((NCHUNK, C), jnp.int32),
            pltpu.VMEM((EW,), jnp.float32),
            [pltpu.VMEM((C, HH), jnp.float32) for _ in range(NBUF)],
            pltpu.VMEM_SHARED((N, HH), jnp.float32),
            [pltpu.SemaphoreType.DMA for _ in range(NBUF)],
        ],
        compiler_params=pltpu.CompilerParams(use_tc_tiling_on_sc=False),
    )
    def k(f_hbm, nbr_hbm, dst_hbm, w_hbm, out_hbm,
          idx_v, dst_v, w_v, bufs, agg_sh, sems):
        stage_v = bufs[0]
        c = lax.axis_index("c")
        s = lax.axis_index("s")

        def zero_row(r, carry):
            for q in range(HH // L):
                stage_v[r, pl.ds(q * L, L)] = jnp.zeros((L,), jnp.float32)
            return carry
        lax.fori_loop(0, OB, zero_row, 0)
        for i in range(pl.cdiv(NBLK, NS)):
            b = s + NS * i

            @pl.when(b < NBLK)
            def _():
                r0 = pl.multiple_of(b * OB, 8)
                pltpu.sync_copy(stage_v, agg_sh.at[pl.ds(r0, OB)])

        row0 = s * NCHUNK
        pltpu.sync_copy(nbr_hbm.at[c, pl.ds(row0, NCHUNK)], idx_v)
        pltpu.sync_copy(dst_hbm.at[pl.ds(row0, NCHUNK)], dst_v)
        pltpu.sync_copy(w_hbm.at[pl.ds(s * EW, EW)], w_v)

        plsc.subcore_barrier()

        for b in range(NBUF):
            pltpu.async_copy(f_hbm.at[idx_v.at[b]], bufs[b], sems[b])

        def outer(it, carry):
            j0 = it * NBUF
            for b in range(NBUF):
                j = j0 + b
                pltpu.make_async_copy(f_hbm.at[idx_v.at[j]], bufs[b], sems[b]).wait()
                for g in range(C // L):
                    w16 = w_v[pl.ds(j * C + g * L, L)]
                    for lane in range(L):
                        e = g * L + lane
                        ws = _bcast_lane(w16, lane)
                        for q in range(HH // L):
                            sl = pl.ds(q * L, L)
                            bufs[b][e, sl] = bufs[b][e, sl] * ws
                pltpu.sync_copy(bufs[b], agg_sh.at[dst_v.at[j]], add=True)

                @pl.when(j + NBUF < NCHUNK)
                def _():
                    pltpu.async_copy(f_hbm.at[idx_v.at[j + NBUF]], bufs[b], sems[b])
            return carry
        lax.fori_loop(0, NCHUNK // NBUF, outer, 0)

        plsc.subcore_barrier()

        for i in range(pl.cdiv(NBLK, NS)):
            b = s + NS * i

            @pl.when(b < NBLK)
            def _():
                r0 = pl.multiple_of(b * OB, 8)
                pltpu.sync_copy(agg_sh.at[pl.ds(r0, OB)], stage_v)
                pltpu.sync_copy(stage_v, out_hbm.at[c, pl.ds(r0, OB)])

    return k(F2, nbr2, dst2d, w)


def _out_ffn(x, p, Wa, Wb, c1, W2, c2):
    blk = 1000

    def body(x_ref, p_ref, wa_ref, wb_ref, c1_ref, w2_ref, c2_ref, o_ref):
        h = jnp.dot(x_ref[...], wa_ref[...], preferred_element_type=jnp.float32)
        h = h + jnp.dot(p_ref[0], wb_ref[...][:HH], preferred_element_type=jnp.float32)
        h = h + jnp.dot(p_ref[1], wb_ref[...][HH:], preferred_element_type=jnp.float32)
        h = _gelu(h + c1_ref[...])
        h = jnp.dot(h, w2_ref[...], preferred_element_type=jnp.float32)
        h = _gelu(h + c2_ref[...])
        nrm = lax.rsqrt(jnp.maximum(jnp.sum(h * h, axis=-1, keepdims=True), 1e-12))
        o_ref[...] = h * nrm

    return pl.pallas_call(
        body,
        grid=(N // blk,),
        in_specs=[
            pl.BlockSpec((blk, D), lambda i: (i, 0)),
            pl.BlockSpec((NC, blk, HH), lambda i: (0, i, 0)),
            pl.BlockSpec((D, H), lambda i: (0, 0)),
            pl.BlockSpec((H, H), lambda i: (0, 0)),
            pl.BlockSpec((1, H), lambda i: (0, 0)),
            pl.BlockSpec((H, H), lambda i: (0, 0)),
            pl.BlockSpec((1, H), lambda i: (0, 0)),
        ],
        out_specs=pl.BlockSpec((blk, H), lambda i: (i, 0)),
        out_shape=jax.ShapeDtypeStruct((N, H), jnp.float32),
    )(x, p, Wa, Wb, c1, W2, c2)


def kernel(node_representations, branches, branch_weights,
           m_g1, m_b1, m_mu1, m_v1, m_W1, m_c1,
           m_g2, m_b2, m_mu2, m_v2, m_W2, m_c2,
           e_g1, e_b1, e_mu1, e_v1, e_W1, e_c1,
           e_g2, e_b2, e_mu2, e_v2, e_W2, e_c2):
    x = node_representations[0]
    dst = branches[0]
    nbr = branches[1]
    w = branch_weights[0, :, 0]

    s1 = m_g1 * lax.rsqrt(m_v1 + 1e-3)
    t1 = m_b1 - m_mu1 * s1
    mW1 = s1[:, None] * m_W1
    mc1 = (m_c1 + t1 @ m_W1)[None]
    s2 = m_g2 * lax.rsqrt(m_v2 + 1e-3)
    t2 = m_b2 - m_mu2 * s2
    mW2 = s2[:, None] * m_W2
    mc2 = (m_c2 + t2 @ m_W2)[None]

    F = _node_ffn(x, mW1, mc1, mW2, mc2)

    zpad_i = jnp.zeros((EP - E,), jnp.int32)
    zpad_f = jnp.zeros((EP - E,), jnp.float32)
    nbr_p = jnp.concatenate([nbr, zpad_i])
    nbr2 = jnp.stack([nbr_p, nbr_p + N]).reshape(NC, EP // C, C)
    dst_p = jnp.concatenate([dst, zpad_i]).reshape(EP // C, C)
    w_p = jnp.concatenate([w, zpad_f])
    F2 = jnp.concatenate([F[:, :HH], F[:, HH:]], axis=0)
    p = _sc_segment_sum(F2, nbr2, dst_p, w_p)

    se = e_g1 * lax.rsqrt(e_v1 + 1e-3)
    te = e_b1 - e_mu1 * se
    eW1 = se[:, None] * e_W1
    ec1 = (e_c1 + te @ e_W1)[None]
    sf = e_g2 * lax.rsqrt(e_v2 + 1e-3)
    tf = e_b2 - e_mu2 * sf
    eW2 = sf[:, None] * e_W2
    ec2 = (e_c2 + tf @ e_W2)[None]

    out = _out_ffn(x, p, eW1[:D], eW1[D:], ec1, eW2, ec2)
    return out[None]


# speedup vs baseline: 6.1162x; 1.0068x over previous
import functools

import jax
import jax.numpy as jnp
from jax import lax
from jax.experimental import pallas as pl
from jax.experimental.pallas import tpu as pltpu
from jax.experimental.pallas import tpu_sc as plsc

N = 10000
E = 320000
D = 128
H = 128

NC = 2
NS = 16
L = 16

HH = H // 2
NBUF = 4
C = 80
NCHUNK = 256
EW = NCHUNK * C
EP = NS * EW
OB = C
NBLK = N // OB


def _gelu(x):
    return x * (0.5 * (1.0 + lax.erf(x * (2.0 ** -0.5))))


def _node_ffn(x, W1, c1, W2, c2):
    blk = 1000

    def body(x_ref, w1_ref, c1_ref, w2_ref, c2_ref, o_ref):
        h = jnp.dot(x_ref[...], w1_ref[...], preferred_element_type=jnp.float32)
        h = _gelu(h + c1_ref[...])
        h = jnp.dot(h, w2_ref[...], preferred_element_type=jnp.float32)
        o_ref[...] = _gelu(h + c2_ref[...])

    return pl.pallas_call(
        body,
        grid=(N // blk,),
        in_specs=[
            pl.BlockSpec((blk, D), lambda i: (i, 0)),
            pl.BlockSpec((D, H), lambda i: (0, 0)),
            pl.BlockSpec((1, H), lambda i: (0, 0)),
            pl.BlockSpec((H, H), lambda i: (0, 0)),
            pl.BlockSpec((1, H), lambda i: (0, 0)),
        ],
        out_specs=pl.BlockSpec((blk, H), lambda i: (i, 0)),
        out_shape=jax.ShapeDtypeStruct((N, H), jnp.float32),
    )(x, W1, c1, W2, c2)


def _bcast_lane(v16, lane):
    idx = jnp.full((L, 1), lane, dtype=jnp.int32)
    return lax.gather(
        v16, idx,
        lax.GatherDimensionNumbers(
            offset_dims=(), collapsed_slice_dims=(0,), start_index_map=(0,)),
        slice_sizes=(1,),
        mode=lax.GatherScatterMode.PROMISE_IN_BOUNDS)


def _sc_segment_sum(F2, nbr2, dst2d, w):
    mesh = plsc.VectorSubcoreMesh(core_axis_name="c", subcore_axis_name="s")

    @functools.partial(
        pl.kernel,
        out_type=jax.ShapeDtypeStruct((NC, N, HH), jnp.float32),
        mesh=mesh,
        scratch_types=[
            pltpu.VMEM((NCHUNK, C), jnp.int32),
            pltpu.VMEM((NCHUNK, C), jnp.int32),
            pltpu.VMEM((EW,), jnp.float32),
            [pltpu.VMEM((C, HH), jnp.float32) for _ in range(NBUF)],
            pltpu.VMEM_SHARED((N, HH), jnp.float32),
            [pltpu.SemaphoreType.DMA for _ in range(NBUF)],
        ],
        compiler_params=pltpu.CompilerParams(use_tc_tiling_on_sc=False),
    )
    def k(f_hbm, nbr_hbm, dst_hbm, w_hbm, out_hbm,
          idx_v, dst_v, w_v, bufs, agg_sh, sems):
        stage_v = bufs[0]
        c = lax.axis_index("c")
        s = lax.axis_index("s")

        def zero_row(r, carry):
            for q in range(HH // L):
                stage_v[r, pl.ds(q * L, L)] = jnp.zeros((L,), jnp.float32)
            return carry
        lax.fori_loop(0, OB, zero_row, 0)
        for i in range(pl.cdiv(NBLK, NS)):
            b = s + NS * i

            @pl.when(b < NBLK)
            def _():
                r0 = pl.multiple_of(b * OB, 8)
                pltpu.sync_copy(stage_v, agg_sh.at[pl.ds(r0, OB)])

        row0 = s * NCHUNK
        pltpu.sync_copy(nbr_hbm.at[c, pl.ds(row0, NCHUNK)], idx_v)
        pltpu.sync_copy(dst_hbm.at[pl.ds(row0, NCHUNK)], dst_v)
        pltpu.sync_copy(w_hbm.at[pl.ds(s * EW, EW)], w_v)

        plsc.subcore_barrier()

        G, S = bufs[0:2], bufs[2:4]
        gs, ss = sems[0:2], sems[2:4]
        for b in range(2):
            pltpu.async_copy(f_hbm.at[idx_v.at[b]], G[b], gs[b])

        def outer(it, carry):
            j0 = it * 2
            for b in range(2):
                j = j0 + b
                pltpu.make_async_copy(f_hbm.at[idx_v.at[j]], G[b], gs[b]).wait()

                @pl.when(j >= 2)
                def _():
                    pltpu.make_async_copy(
                        S[b], agg_sh.at[dst_v.at[j - 2]], ss[b]).wait()

                for g in range(C // L):
                    w16 = w_v[pl.ds(j * C + g * L, L)]
                    for lane in range(L):
                        e = g * L + lane
                        ws = _bcast_lane(w16, lane)
                        for q in range(HH // L):
                            sl = pl.ds(q * L, L)
                            S[b][e, sl] = G[b][e, sl] * ws

                pltpu.async_copy(S[b], agg_sh.at[dst_v.at[j]], ss[b], add=True)

                @pl.when(j + 2 < NCHUNK)
                def _():
                    pltpu.async_copy(f_hbm.at[idx_v.at[j + 2]], G[b], gs[b])
            return carry
        lax.fori_loop(0, NCHUNK // 2, outer, 0)

        for b in range(2):
            pltpu.make_async_copy(
                S[b], agg_sh.at[dst_v.at[NCHUNK - 2 + b]], ss[b]).wait()

        plsc.subcore_barrier()

        for i in range(pl.cdiv(NBLK, NS)):
            b = s + NS * i

            @pl.when(b < NBLK)
            def _():
                r0 = pl.multiple_of(b * OB, 8)
                pltpu.sync_copy(agg_sh.at[pl.ds(r0, OB)], stage_v)
                pltpu.sync_copy(stage_v, out_hbm.at[c, pl.ds(r0, OB)])

    return k(F2, nbr2, dst2d, w)


def _out_ffn(x, p, Wa, Wb, c1, W2, c2):
    blk = 1000

    def body(x_ref, p_ref, wa_ref, wb_ref, c1_ref, w2_ref, c2_ref, o_ref):
        h = jnp.dot(x_ref[...], wa_ref[...], preferred_element_type=jnp.float32)
        h = h + jnp.dot(p_ref[0], wb_ref[...][:HH], preferred_element_type=jnp.float32)
        h = h + jnp.dot(p_ref[1], wb_ref[...][HH:], preferred_element_type=jnp.float32)
        h = _gelu(h + c1_ref[...])
        h = jnp.dot(h, w2_ref[...], preferred_element_type=jnp.float32)
        h = _gelu(h + c2_ref[...])
        nrm = lax.rsqrt(jnp.maximum(jnp.sum(h * h, axis=-1, keepdims=True), 1e-12))
        o_ref[...] = h * nrm

    return pl.pallas_call(
        body,
        grid=(N // blk,),
        in_specs=[
            pl.BlockSpec((blk, D), lambda i: (i, 0)),
            pl.BlockSpec((NC, blk, HH), lambda i: (0, i, 0)),
            pl.BlockSpec((D, H), lambda i: (0, 0)),
            pl.BlockSpec((H, H), lambda i: (0, 0)),
            pl.BlockSpec((1, H), lambda i: (0, 0)),
            pl.BlockSpec((H, H), lambda i: (0, 0)),
            pl.BlockSpec((1, H), lambda i: (0, 0)),
        ],
        out_specs=pl.BlockSpec((blk, H), lambda i: (i, 0)),
        out_shape=jax.ShapeDtypeStruct((N, H), jnp.float32),
    )(x, p, Wa, Wb, c1, W2, c2)


def kernel(node_representations, branches, branch_weights,
           m_g1, m_b1, m_mu1, m_v1, m_W1, m_c1,
           m_g2, m_b2, m_mu2, m_v2, m_W2, m_c2,
           e_g1, e_b1, e_mu1, e_v1, e_W1, e_c1,
           e_g2, e_b2, e_mu2, e_v2, e_W2, e_c2):
    x = node_representations[0]
    dst = branches[0]
    nbr = branches[1]
    w = branch_weights[0, :, 0]

    s1 = m_g1 * lax.rsqrt(m_v1 + 1e-3)
    t1 = m_b1 - m_mu1 * s1
    mW1 = s1[:, None] * m_W1
    mc1 = (m_c1 + t1 @ m_W1)[None]
    s2 = m_g2 * lax.rsqrt(m_v2 + 1e-3)
    t2 = m_b2 - m_mu2 * s2
    mW2 = s2[:, None] * m_W2
    mc2 = (m_c2 + t2 @ m_W2)[None]

    F = _node_ffn(x, mW1, mc1, mW2, mc2)

    zpad_i = jnp.zeros((EP - E,), jnp.int32)
    zpad_f = jnp.zeros((EP - E,), jnp.float32)
    nbr_p = jnp.concatenate([nbr, zpad_i])
    nbr2 = jnp.stack([nbr_p, nbr_p + N]).reshape(NC, EP // C, C)
    dst_p = jnp.concatenate([dst, zpad_i]).reshape(EP // C, C)
    w_p = jnp.concatenate([w, zpad_f])
    F2 = jnp.concatenate([F[:, :HH], F[:, HH:]], axis=0)
    p = _sc_segment_sum(F2, nbr2, dst_p, w_p)

    se = e_g1 * lax.rsqrt(e_v1 + 1e-3)
    te = e_b1 - e_mu1 * se
    eW1 = se[:, None] * e_W1
    ec1 = (e_c1 + te @ e_W1)[None]
    sf = e_g2 * lax.rsqrt(e_v2 + 1e-3)
    tf = e_b2 - e_mu2 * sf
    eW2 = sf[:, None] * e_W2
    ec2 = (e_c2 + tf @ e_W2)[None]

    out = _out_ffn(x, p, eW1[:D], eW1[D:], ec1, eW2, ec2)
    return out[None]


# speedup vs baseline: 10.8064x; 1.7668x over previous
import functools

import jax
import jax.numpy as jnp
from jax import lax
from jax.experimental import pallas as pl
from jax.experimental.pallas import tpu as pltpu
from jax.experimental.pallas import tpu_sc as plsc

N = 10000
E = 320000
D = 128
H = 128

NC = 2
NS = 16
L = 16

HH = H // 2
NBUF = 4
C = 80
EW = E // NS
NCHUNK = EW // C
OB = C
NBLK = N // OB


def _gelu(x):
    return x * (0.5 * (1.0 + lax.erf(x * (2.0 ** -0.5))))


def _node_ffn(x, W1, c1, W2, c2):
    blk = 1000

    def body(x_ref, w1_ref, c1_ref, w2_ref, c2_ref, o_ref):
        h = jnp.dot(x_ref[...], w1_ref[...], preferred_element_type=jnp.float32)
        h = _gelu(h + c1_ref[...])
        h = jnp.dot(h, w2_ref[...], preferred_element_type=jnp.float32)
        h = _gelu(h + c2_ref[...])
        o_ref[0] = h[:, :HH]
        o_ref[1] = h[:, HH:]

    return pl.pallas_call(
        body,
        grid=(N // blk,),
        in_specs=[
            pl.BlockSpec((blk, D), lambda i: (i, 0)),
            pl.BlockSpec((D, H), lambda i: (0, 0)),
            pl.BlockSpec((1, H), lambda i: (0, 0)),
            pl.BlockSpec((H, H), lambda i: (0, 0)),
            pl.BlockSpec((1, H), lambda i: (0, 0)),
        ],
        out_specs=pl.BlockSpec((2, blk, HH), lambda i: (0, i, 0)),
        out_shape=jax.ShapeDtypeStruct((2, N, HH), jnp.float32),
    )(x, W1, c1, W2, c2)


def _bcast_lane(v16, lane):
    idx = jnp.full((L, 1), lane, dtype=jnp.int32)
    return lax.gather(
        v16, idx,
        lax.GatherDimensionNumbers(
            offset_dims=(), collapsed_slice_dims=(0,), start_index_map=(0,)),
        slice_sizes=(1,),
        mode=lax.GatherScatterMode.PROMISE_IN_BOUNDS)


def _sc_segment_sum(F2, nbr2d, dst2d, w):
    mesh = plsc.VectorSubcoreMesh(core_axis_name="c", subcore_axis_name="s")

    @functools.partial(
        pl.kernel,
        out_type=jax.ShapeDtypeStruct((NC, N, HH), jnp.float32),
        mesh=mesh,
        scratch_types=[
            pltpu.VMEM((NCHUNK, C), jnp.int32),
            pltpu.VMEM((NCHUNK, C), jnp.int32),
            pltpu.VMEM((EW,), jnp.float32),
            [pltpu.VMEM((C, HH), jnp.float32) for _ in range(NBUF)],
            pltpu.VMEM_SHARED((N, HH), jnp.float32),
            [pltpu.SemaphoreType.DMA for _ in range(NBUF)],
        ],
        compiler_params=pltpu.CompilerParams(use_tc_tiling_on_sc=False),
    )
    def k(f_hbm, nbr_hbm, dst_hbm, w_hbm, out_hbm,
          idx_v, dst_v, w_v, bufs, agg_sh, sems):
        stage_v = bufs[0]
        c = lax.axis_index("c")
        s = lax.axis_index("s")

        def zero_row(r, carry):
            for q in range(HH // L):
                stage_v[r, pl.ds(q * L, L)] = jnp.zeros((L,), jnp.float32)
            return carry
        lax.fori_loop(0, OB, zero_row, 0)
        for i in range(pl.cdiv(NBLK, NS)):
            b = s + NS * i

            @pl.when(b < NBLK)
            def _():
                r0 = pl.multiple_of(b * OB, 8)
                pltpu.sync_copy(stage_v, agg_sh.at[pl.ds(r0, OB)])

        row0 = s * NCHUNK
        pltpu.sync_copy(nbr_hbm.at[pl.ds(row0, NCHUNK)], idx_v)
        pltpu.sync_copy(dst_hbm.at[pl.ds(row0, NCHUNK)], dst_v)
        pltpu.sync_copy(w_hbm.at[pl.ds(s * EW, EW)], w_v)

        cN = c * N

        def shift_row(r, carry):
            for g in range(C // L):
                sl = pl.ds(g * L, L)
                idx_v[r, sl] = idx_v[r, sl] + cN
            return carry
        lax.fori_loop(0, NCHUNK, shift_row, 0)

        plsc.subcore_barrier()

        G, S = bufs[0:2], bufs[2:4]
        gs, ss = sems[0:2], sems[2:4]
        for b in range(2):
            pltpu.async_copy(f_hbm.at[idx_v.at[b]], G[b], gs[b])

        def outer(it, carry):
            j0 = it * 2
            for b in range(2):
                j = j0 + b
                pltpu.make_async_copy(f_hbm.at[idx_v.at[j]], G[b], gs[b]).wait()

                @pl.when(j >= 2)
                def _():
                    pltpu.make_async_copy(
                        S[b], agg_sh.at[dst_v.at[j - 2]], ss[b]).wait()

                for g in range(C // L):
                    w16 = w_v[pl.ds(j * C + g * L, L)]
                    for lane in range(L):
                        e = g * L + lane
                        ws = _bcast_lane(w16, lane)
                        for q in range(HH // L):
                            sl = pl.ds(q * L, L)
                            S[b][e, sl] = G[b][e, sl] * ws

                pltpu.async_copy(S[b], agg_sh.at[dst_v.at[j]], ss[b], add=True)

                @pl.when(j + 2 < NCHUNK)
                def _():
                    pltpu.async_copy(f_hbm.at[idx_v.at[j + 2]], G[b], gs[b])
            return carry
        lax.fori_loop(0, NCHUNK // 2, outer, 0)

        for b in range(2):
            pltpu.make_async_copy(
                S[b], agg_sh.at[dst_v.at[NCHUNK - 2 + b]], ss[b]).wait()

        plsc.subcore_barrier()

        for i in range(pl.cdiv(NBLK, NS)):
            b = s + NS * i

            @pl.when(b < NBLK)
            def _():
                r0 = pl.multiple_of(b * OB, 8)
                pltpu.sync_copy(agg_sh.at[pl.ds(r0, OB)], stage_v)
                pltpu.sync_copy(stage_v, out_hbm.at[c, pl.ds(r0, OB)])

    return k(F2, nbr2d, dst2d, w)


def _out_ffn(x, p, Wa, Wb, c1, W2, c2):
    blk = 1000

    def body(x_ref, p_ref, wa_ref, wb_ref, c1_ref, w2_ref, c2_ref, o_ref):
        h = jnp.dot(x_ref[...], wa_ref[...], preferred_element_type=jnp.float32)
        h = h + jnp.dot(p_ref[0], wb_ref[...][:HH], preferred_element_type=jnp.float32)
        h = h + jnp.dot(p_ref[1], wb_ref[...][HH:], preferred_element_type=jnp.float32)
        h = _gelu(h + c1_ref[...])
        h = jnp.dot(h, w2_ref[...], preferred_element_type=jnp.float32)
        h = _gelu(h + c2_ref[...])
        nrm = lax.rsqrt(jnp.maximum(jnp.sum(h * h, axis=-1, keepdims=True), 1e-12))
        o_ref[...] = h * nrm

    return pl.pallas_call(
        body,
        grid=(N // blk,),
        in_specs=[
            pl.BlockSpec((blk, D), lambda i: (i, 0)),
            pl.BlockSpec((NC, blk, HH), lambda i: (0, i, 0)),
            pl.BlockSpec((D, H), lambda i: (0, 0)),
            pl.BlockSpec((H, H), lambda i: (0, 0)),
            pl.BlockSpec((1, H), lambda i: (0, 0)),
            pl.BlockSpec((H, H), lambda i: (0, 0)),
            pl.BlockSpec((1, H), lambda i: (0, 0)),
        ],
        out_specs=pl.BlockSpec((blk, H), lambda i: (i, 0)),
        out_shape=jax.ShapeDtypeStruct((N, H), jnp.float32),
    )(x, p, Wa, Wb, c1, W2, c2)


def kernel(node_representations, branches, branch_weights,
           m_g1, m_b1, m_mu1, m_v1, m_W1, m_c1,
           m_g2, m_b2, m_mu2, m_v2, m_W2, m_c2,
           e_g1, e_b1, e_mu1, e_v1, e_W1, e_c1,
           e_g2, e_b2, e_mu2, e_v2, e_W2, e_c2):
    x = node_representations[0]
    dst = branches[0]
    nbr = branches[1]
    w = branch_weights[0, :, 0]

    s1 = m_g1 * lax.rsqrt(m_v1 + 1e-3)
    t1 = m_b1 - m_mu1 * s1
    mW1 = s1[:, None] * m_W1
    mc1 = (m_c1 + t1 @ m_W1)[None]
    s2 = m_g2 * lax.rsqrt(m_v2 + 1e-3)
    t2 = m_b2 - m_mu2 * s2
    mW2 = s2[:, None] * m_W2
    mc2 = (m_c2 + t2 @ m_W2)[None]

    F2 = _node_ffn(x, mW1, mc1, mW2, mc2).reshape(2 * N, HH)

    p = _sc_segment_sum(F2, nbr.reshape(E // C, C), dst.reshape(E // C, C), w)

    se = e_g1 * lax.rsqrt(e_v1 + 1e-3)
    te = e_b1 - e_mu1 * se
    eW1 = se[:, None] * e_W1
    ec1 = (e_c1 + te @ e_W1)[None]
    sf = e_g2 * lax.rsqrt(e_v2 + 1e-3)
    tf = e_b2 - e_mu2 * sf
    eW2 = sf[:, None] * e_W2
    ec2 = (e_c2 + tf @ e_W2)[None]

    out = _out_ffn(x, p, eW1[:D], eW1[D:], ec1, eW2, ec2)
    return out[None]
